# Initial kernel scaffold; baseline (speedup 1.0000x reference)
import jax
import jax.numpy as jnp
from jax.experimental import pallas as pl


def kernel(node_features, edge_features, params, from_idx, to_idx, from_local, to_local):
    raise NotImplementedError("write your pallas kernel here")


# speedup vs baseline: 203.0233x; 203.0233x over previous
import jax
import jax.numpy as jnp
from jax.experimental import pallas as pl
from jax.experimental.pallas import tpu as pltpu

_B = 32
_MAXN = 64
_EG = 256
_D = 64
_M = 64
_EE = 16
_P = 3
_T = 2
_SK_ITERS = 10
_SK_TEMP = 0.1

_f32 = jnp.float32


def _mm(a, b):
    return jax.lax.dot_general(a, b, (((1,), (0,)), ((), ())),
                               preferred_element_type=_f32)


def _mm0(a, b):
    return jax.lax.dot_general(a, b, (((0,), (0,)), ((), ())),
                               preferred_element_type=_f32)


def _mmr(a, b):
    return jax.lax.dot_general(a, b, (((1,), (1,)), ((), ())),
                               preferred_element_type=_f32)


def _lse(x, axis):
    m = jnp.max(x, axis=axis, keepdims=True)
    return m + jnp.log(jnp.sum(jnp.exp(x - m), axis=axis, keepdims=True))


def _sink(la):
    la = la / _SK_TEMP
    for _ in range(_SK_ITERS):
        la = la - _lse(la, 1)
        la = la - _lse(la, 0)
    return jnp.exp(la)


def _body(nf_ref, ef_ref, pfrom_ref, pto_ref,
          enW_r, enb_r, eeW_r, eeb_r,
          mW1_r, mb1_r, mW2_r, mb2_r,
          uW1_r, ub1_r, uW2_r, ub2_r,
          nW1_r, nb1_r, nW2_r, nb2_r,
          iW1_r, ib1_r, iW2_r, ib2_r,
          sW1_r, sb1_r, sW2_r, sb2_r,
          out_ref):
    nf = nf_ref[0]
    ef = ef_ref[0]
    enW, enb = enW_r[...], enb_r[...]
    eeW, eeb = eeW_r[...], eeb_r[...]
    mW1, mb1, mW2, mb2 = mW1_r[...], mb1_r[...], mW2_r[...], mb2_r[...]
    uW1, ub1, uW2, ub2 = uW1_r[...], ub1_r[...], uW2_r[...], ub2_r[...]
    nW1, nb1, nW2, nb2 = nW1_r[...], nb1_r[...], nW2_r[...], nb2_r[...]
    iW1, ib1, iW2, ib2 = iW1_r[...], ib1_r[...], iW2_r[...], ib2_r[...]
    sW1, sb1, sW2, sb2 = sW1_r[...], sb1_r[...], sW2_r[...], sb2_r[...]

    enc_n = _mm(nf, enW) + enb
    enc_e = _mm(ef, eeW) + eeb

    pfi = pfrom_ref[0]
    pti = pto_ref[0]
    rows = jax.lax.broadcasted_iota(jnp.int32, (128, 512), 0)
    OfT = (rows == pfi).astype(_f32)
    OtT = (rows == pti).astype(_f32)

    def ni_mlp(x, inter):
        h = jnp.maximum(_mm(x, nW1[0:64]) + _mm(inter, nW1[64:128]) + nb1, 0.0)
        return _mm(h, nW2) + nb2

    def ei_mlp(e, einter):
        h = jnp.maximum(_mm(e, iW1[0:16]) + _mm(einter, iW1[16:80]) + ib1, 0.0)
        return _mm(h, iW2) + ib2

    def msg_mlp(a, b, e):
        h = jnp.maximum(_mm(a, mW1[0:64]) + _mm(b, mW1[64:128])
                        + _mm(e, mW1[128:144]) + mb1, 0.0)
        return _mm(h, mW2) + mb2

    def upd_mlp(nc, agg):
        h = jnp.maximum(_mm(nc, uW1[0:64]) + _mm(agg, uW1[64:128]) + ub1, 0.0)
        return _mm(h, uW2) + ub2

    def sk_mlp(x):
        h = jnp.maximum(_mm(x, sW1) + sb1, 0.0)
        return _mm(h, sW2) + sb2

    zn = jnp.zeros((128, 64), _f32)
    ze = jnp.zeros((512, 64), _f32)
    node_blks = [zn, zn, zn, zn]
    edge_blks = [ze, ze, ze, ze]
    ffq = ffc = plan = None

    for t in range(_T):
        nfe, efe = enc_n, enc_e
        nc = ni_mlp(nfe, node_blks[0])
        new_node = [zn]
        new_edge = [ze]
        for s in range(1, _P + 1):
            ec = ei_mlp(efe, edge_blks[s - 1])
            ncf = _mm0(OfT, nc)
            nct = _mm0(OtT, nc)
            m1 = msg_mlp(ncf, nct, ec)
            m2 = msg_mlp(nct, ncf, ec)
            agg = _mm(OtT, m1) + _mm(OfT, m2)
            nfe = upd_mlp(nc, agg)
            nc = ni_mlp(nfe, node_blks[s])
            new_node.append(nfe)
            if t < _T - 1:
                ncf2 = _mm0(OfT, nc)
                nct2 = _mm0(OtT, nc)
                new_edge.append(msg_mlp(ncf2, nct2, ec))
        ffq = new_node[_P][0:64]
        ffc = new_node[_P][64:128]
        tqf = sk_mlp(ffq)
        tcf = sk_mlp(ffc)
        plan = _sink(_mmr(tqf, tcf))
        if t < _T - 1:
            node_blks = [zn] + [
                jnp.concatenate([_mm(plan, blk[64:128]), _mm0(plan, blk[0:64])],
                                axis=0)
                for blk in new_node[1:]
            ]
            OfqT, OfcT = OfT[0:64, 0:256], OfT[64:128, 256:512]
            OtqT, OtcT = OtT[0:64, 0:256], OtT[64:128, 256:512]
            rf = _mm0(OfqT, plan)
            rt = _mm0(OtqT, plan)
            straight = _mm(rf, OfcT) * _mm(rt, OtcT)
            cross = _mm(rf, OtcT) * _mm(rt, OfcT)
            eplan = _sink(straight + cross)
            edge_blks = [ze] + [
                jnp.concatenate([_mm(eplan, blk[256:512]),
                                 _mm0(eplan, blk[0:256])], axis=0)
                for blk in new_edge[1:]
            ]

    score = -jnp.sum(jnp.maximum(ffq - _mm(plan, ffc), 0.0))
    out_ref[...] = jnp.full((1, 8, 128), score, _f32)


def kernel(node_features, edge_features, params, from_idx, to_idx,
           from_local, to_local):
    p = params
    nf = node_features.reshape(_B, 2 * _MAXN, node_features.shape[-1])
    ef = edge_features.reshape(_B, 2 * _EG, edge_features.shape[-1])
    off = jnp.array([0, _MAXN], jnp.int32).reshape(1, 2, 1)
    pfrom = (from_local.reshape(_B, 2, _EG) + off).reshape(_B, 1, 2 * _EG)
    pto = (to_local.reshape(_B, 2, _EG) + off).reshape(_B, 1, 2 * _EG)

    def row(v):
        return v.reshape(1, -1)

    weights = [
        p['enc_node_W'], row(p['enc_node_b']),
        p['enc_edge_W'], row(p['enc_edge_b']),
        p['msg_W1'], row(p['msg_b1']), p['msg_W2'], row(p['msg_b2']),
        p['upd_W1'], row(p['upd_b1']), p['upd_W2'], row(p['upd_b2']),
        p['ni_W1'], row(p['ni_b1']), p['ni_W2'], row(p['ni_b2']),
        p['ei_W1'], row(p['ei_b1']), p['ei_W2'], row(p['ei_b2']),
        p['sk_W1'], row(p['sk_b1']), p['sk_W2'], row(p['sk_b2']),
    ]

    def pair_spec(shape):
        nd = len(shape)
        return pl.BlockSpec((1,) + shape[1:],
                            lambda b, _n=nd: (b,) + (0,) * (_n - 1))

    def full_spec(shape):
        nd = len(shape)
        return pl.BlockSpec(shape, lambda b, _n=nd: (0,) * _n)

    in_specs = [pair_spec(nf.shape), pair_spec(ef.shape),
                pair_spec(pfrom.shape), pair_spec(pto.shape)]
    in_specs += [full_spec(w.shape) for w in weights]

    out = pl.pallas_call(
        _body,
        grid=(_B,),
        in_specs=in_specs,
        out_specs=pl.BlockSpec((1, 8, 128), lambda b: (b, 0, 0)),
        out_shape=jax.ShapeDtypeStruct((_B, 8, 128), _f32),
        compiler_params=pltpu.CompilerParams(
            dimension_semantics=("arbitrary",)),
    )(nf, ef, pfrom, pto, *weights)
    return out[:, 0, 0]
